# Initial kernel scaffold; baseline (speedup 1.0000x reference)
import jax
import jax.numpy as jnp
from jax.experimental import pallas as pl


def kernel(class_logits, box_regression, corners_semantic, proposals):
    raise NotImplementedError("write your pallas kernel here")


# speedup vs baseline: 15.2561x; 15.2561x over previous
import functools

import jax
import jax.numpy as jnp
import numpy as np
from jax import lax
from jax.experimental import pallas as pl
from jax.experimental.pallas import tpu as pltpu
from jax.experimental.pallas import tpu_sc as plsc

_N = 20000
_C = 10
_NCLS = _C - 1
_SCORE_THRESH = 0.05
_NMS_THRESH = 0.5
_DET = 100
_TOP = 512
_CLIP = float(np.log(1000.0 / 16.0))
_W = _TOP // 32


def _nms_sc_body(sup_hbm, valid_hbm, out_hbm, sup_v, keep_v):
    nc = 2
    wid = lax.axis_index("s") * nc + lax.axis_index("c")

    @pl.when(wid < _NCLS)
    def _():
        pltpu.sync_copy(sup_hbm.at[wid], sup_v)
        pltpu.sync_copy(valid_hbm.at[wid], keep_v)

        dnums = lax.GatherDimensionNumbers(
            offset_dims=(), collapsed_slice_dims=(0,), start_index_map=(0,)
        )

        def body(i, keep):
            w = lax.shift_right_logical(i, 5)
            b = lax.bitwise_and(i, 31)
            w_vec = jnp.full((16,), w, jnp.int32)
            word = lax.gather(
                keep,
                w_vec[:, None],
                dimension_numbers=dnums,
                slice_sizes=(1,),
                mode=lax.GatherScatterMode.PROMISE_IN_BOUNDS,
            )
            b_vec = jnp.full((16,), b, jnp.int32)
            bit = lax.bitwise_and(lax.shift_right_logical(word, b_vec), 1)
            mask = jnp.where(bit == 1, -1, 0).astype(jnp.int32)
            row = sup_v[pl.ds(i * _W, _W)]
            return lax.bitwise_and(
                keep, lax.bitwise_not(lax.bitwise_and(row, mask))
            )

        keep_v[:] = lax.fori_loop(0, _TOP, body, keep_v[:])
        pltpu.sync_copy(keep_v, out_hbm.at[wid])


@jax.jit
def _run_nms(sup_words, valid_words):
    mesh = plsc.VectorSubcoreMesh(core_axis_name="c", subcore_axis_name="s")
    f = pl.kernel(
        _nms_sc_body,
        out_type=jax.ShapeDtypeStruct((_NCLS, _W), jnp.int32),
        scratch_types=[
            pltpu.VMEM((_TOP * _W,), jnp.int32),
            pltpu.VMEM((_W,), jnp.int32),
        ],
        mesh=mesh,
    )
    return f(sup_words, valid_words)


def _pack_bits(bits):
    shape = bits.shape[:-1] + (bits.shape[-1] // 32, 32)
    weights = jnp.left_shift(
        jnp.uint32(1), jnp.arange(32, dtype=jnp.uint32)
    )
    words = jnp.sum(bits.reshape(shape).astype(jnp.uint32) * weights, axis=-1)
    return lax.bitcast_convert_type(words, jnp.int32)


def kernel(class_logits, box_regression, corners_semantic, proposals):
    probs = jax.nn.softmax(class_logits, axis=-1)
    s = probs[:, 1:].T
    s_masked = jnp.where(s > _SCORE_THRESH, s, -1.0)
    top_s, top_i = lax.top_k(s_masked, _TOP)

    props = proposals[top_i]
    reg_all = box_regression.reshape(_N, _C, 7)
    cls_idx = jnp.arange(1, _C, dtype=jnp.int32)[:, None]
    reg = reg_all[top_i, cls_idx]
    ctr = props[..., 0:3]
    size = jnp.abs(props[..., 3:6]) + 1e-3
    theta = props[..., 6:7]
    pred_ctr = (reg[..., 0:3] / 10.0) * size + ctr
    pred_size = jnp.exp(jnp.minimum(reg[..., 3:6] / 5.0, _CLIP)) * size
    pred_theta = theta + reg[..., 6:7]
    boxes = jnp.concatenate([pred_ctr, pred_size, pred_theta], axis=-1)

    x1 = boxes[..., 0] - boxes[..., 3] / 2.0
    x2 = boxes[..., 0] + boxes[..., 3] / 2.0
    y1 = boxes[..., 1] - boxes[..., 4] / 2.0
    y2 = boxes[..., 1] + boxes[..., 4] / 2.0
    z1 = boxes[..., 2]
    z2 = boxes[..., 2] + boxes[..., 5]

    def inter(a1, a2):
        lo = jnp.maximum(a1[:, :, None], a1[:, None, :])
        hi = jnp.minimum(a2[:, :, None], a2[:, None, :])
        return jnp.clip(hi - lo, 0.0)

    iv = inter(x1, x2) * inter(y1, y2) * inter(z1, z2)
    vol = (
        jnp.clip(x2 - x1, 0.0)
        * jnp.clip(y2 - y1, 0.0)
        * jnp.clip(z2 - z1, 0.0)
    )
    union = vol[:, :, None] + vol[:, None, :] - iv
    iou = iv / jnp.maximum(union, 1e-8)
    col_gt_row = jnp.arange(_TOP)[None, :] > jnp.arange(_TOP)[:, None]
    sup = (iou > _NMS_THRESH) & col_gt_row[None]
    sup_words = _pack_bits(sup).reshape(_NCLS, _TOP * _W)

    valid_words = _pack_bits(top_s > _SCORE_THRESH)

    keep_words = _run_nms(sup_words, valid_words)
    keep = (
        jnp.right_shift(
            lax.bitcast_convert_type(keep_words, jnp.uint32)[:, :, None],
            jnp.arange(32, dtype=jnp.uint32)[None, None, :],
        )
        & 1
    ).astype(bool).reshape(_NCLS, _TOP)

    s_final = jnp.where(keep, top_s, -1.0)
    scores_cat = s_final.reshape(-1)
    boxes_cat = boxes.reshape(-1, 7)
    labels_cat = jnp.repeat(
        jnp.arange(1, _C, dtype=jnp.int32), _TOP, total_repeat_length=_NCLS * _TOP
    )
    final_s, final_idx = lax.top_k(scores_cat, _DET)
    return boxes_cat[final_idx], final_s, labels_cat[final_idx]
